# baseline (speedup 1.0000x reference)
import functools

import jax
import jax.numpy as jnp
import numpy as np
from jax import lax
from jax.experimental import pallas as pl
from jax.experimental.pallas import tpu as pltpu
from jax.experimental.pallas import tpu_sc as plsc

_NC = 2
_NS = 16
_NW = _NC * _NS

_B = 4
_R = 4096
_D = 1024
_ROWS = _B * _R
_RPW = _ROWS // _NW
_C = 8
_NCHUNK = _RPW // _C
_NBUF = 12


@functools.partial(
    pl.kernel,
    out_type=jax.ShapeDtypeStruct((_ROWS, _D), jnp.float32),
    mesh=plsc.VectorSubcoreMesh(
        core_axis_name="c", subcore_axis_name="s",
        num_cores=_NC, num_subcores=_NS),
    scratch_types=[
        pltpu.VMEM((_RPW,), jnp.int32),
        pltpu.VMEM((_NBUF, _C, _D), jnp.float32),
    ] + [pltpu.SemaphoreType.DMA] * (2 * _NBUF),
)
def _shuffle_sc(x_hbm, idx_hbm, out_hbm, idx_v, rows_v, *sems):
    gsem = sems[:_NBUF]
    ssem = sems[_NBUF:]
    wid = lax.axis_index("s") * _NC + lax.axis_index("c")
    base = wid * _RPW
    pltpu.sync_copy(idx_hbm.at[pl.ds(base, _RPW)], idx_v)

    def gather(ci):
        slot = ci % _NBUF
        return pltpu.make_async_copy(
            x_hbm.at[idx_v.at[pl.ds(ci * _C, _C)]], rows_v.at[slot],
            gsem[slot])

    def scatter(ci):
        slot = ci % _NBUF
        return pltpu.make_async_copy(
            rows_v.at[slot], out_hbm.at[pl.ds(base + ci * _C, _C)],
            ssem[slot])

    for ci in range(_NBUF - 1):
        gather(ci).start()
    for ci in range(_NCHUNK):
        gather(ci).wait()
        scatter(ci).start()
        nxt = ci + _NBUF - 1
        if nxt < _NCHUNK:
            if ci >= 1:
                scatter(ci - 1).wait()
            gather(nxt).start()
    for ci in range(_NCHUNK - _NBUF, _NCHUNK):
        scatter(ci).wait()


def _flat_index_vector(num_rows):
    perm = np.asarray(jax.random.permutation(jax.random.key(42), num_rows))
    idx = perm[None, :] + np.arange(_B)[:, None] * num_rows
    return idx.reshape(-1).astype(np.int32)


_IDX = _flat_index_vector(_R)


def kernel(x):
    idx = jnp.asarray(_IDX)
    flat = x.reshape(_ROWS, _D)
    out = _shuffle_sc(flat, idx)
    return out.reshape(_B, _R, _D)


# speedup vs baseline: 1.0141x; 1.0141x over previous
import functools

import jax
import jax.numpy as jnp
import numpy as np
from jax import lax
from jax.experimental import pallas as pl
from jax.experimental.pallas import tpu as pltpu
from jax.experimental.pallas import tpu_sc as plsc

_NC = 2
_NS = 16
_NW = _NC * _NS

_B = 4
_R = 4096
_D = 1024
_ROWS = _B * _R
_RPW = _ROWS // _NW
_C = 16
_NCHUNK = _RPW // _C
_NBUF = 6


@functools.partial(
    pl.kernel,
    out_type=jax.ShapeDtypeStruct((_ROWS, _D), jnp.float32),
    mesh=plsc.VectorSubcoreMesh(
        core_axis_name="c", subcore_axis_name="s",
        num_cores=_NC, num_subcores=_NS),
    scratch_types=[
        pltpu.VMEM((_RPW,), jnp.int32),
        pltpu.VMEM((_NBUF, _C, _D), jnp.float32),
    ] + [pltpu.SemaphoreType.DMA] * (2 * _NBUF),
)
def _shuffle_sc(x_hbm, idx_hbm, out_hbm, idx_v, rows_v, *sems):
    gsem = sems[:_NBUF]
    ssem = sems[_NBUF:]
    wid = lax.axis_index("s") * _NC + lax.axis_index("c")
    base = wid * _RPW
    pltpu.sync_copy(idx_hbm.at[pl.ds(base, _RPW)], idx_v)

    def gather(ci):
        slot = ci % _NBUF
        return pltpu.make_async_copy(
            x_hbm.at[idx_v.at[pl.ds(ci * _C, _C)]], rows_v.at[slot],
            gsem[slot])

    def scatter(ci):
        slot = ci % _NBUF
        return pltpu.make_async_copy(
            rows_v.at[slot], out_hbm.at[pl.ds(base + ci * _C, _C)],
            ssem[slot])

    for ci in range(_NBUF - 1):
        gather(ci).start()
    for ci in range(_NCHUNK):
        gather(ci).wait()
        scatter(ci).start()
        nxt = ci + _NBUF - 1
        if nxt < _NCHUNK:
            if ci >= 1:
                scatter(ci - 1).wait()
            gather(nxt).start()
    for ci in range(_NCHUNK - _NBUF, _NCHUNK):
        scatter(ci).wait()


def _flat_index_vector(num_rows):
    perm = np.asarray(jax.random.permutation(jax.random.key(42), num_rows))
    idx = perm[None, :] + np.arange(_B)[:, None] * num_rows
    return idx.reshape(-1).astype(np.int32)


_IDX = _flat_index_vector(_R)


def kernel(x):
    idx = jnp.asarray(_IDX)
    flat = x.reshape(_ROWS, _D)
    out = _shuffle_sc(flat, idx)
    return out.reshape(_B, _R, _D)


# speedup vs baseline: 1.0144x; 1.0004x over previous
import functools

import jax
import jax.numpy as jnp
import numpy as np
from jax import lax
from jax.experimental import pallas as pl
from jax.experimental.pallas import tpu as pltpu
from jax.experimental.pallas import tpu_sc as plsc

_NC = 2
_NS = 16
_NW = _NC * _NS

_B = 4
_R = 4096
_D = 1024
_ROWS = _B * _R
_RPW = _ROWS // _NW
_C = 16
_NCHUNK = _RPW // _C
_NBUF = 7


@functools.partial(
    pl.kernel,
    out_type=jax.ShapeDtypeStruct((_ROWS, _D), jnp.float32),
    mesh=plsc.VectorSubcoreMesh(
        core_axis_name="c", subcore_axis_name="s",
        num_cores=_NC, num_subcores=_NS),
    scratch_types=[
        pltpu.VMEM((_RPW,), jnp.int32),
        pltpu.VMEM((_NBUF, _C, _D), jnp.float32),
    ] + [pltpu.SemaphoreType.DMA] * (2 * _NBUF),
)
def _shuffle_sc(x_hbm, idx_hbm, out_hbm, idx_v, rows_v, *sems):
    gsem = sems[:_NBUF]
    ssem = sems[_NBUF:]
    wid = lax.axis_index("s") * _NC + lax.axis_index("c")
    base = wid * _RPW
    pltpu.sync_copy(idx_hbm.at[pl.ds(base, _RPW)], idx_v)

    def gather(ci):
        slot = ci % _NBUF
        return pltpu.make_async_copy(
            x_hbm.at[idx_v.at[pl.ds(ci * _C, _C)]], rows_v.at[slot],
            gsem[slot])

    def scatter(ci):
        slot = ci % _NBUF
        return pltpu.make_async_copy(
            rows_v.at[slot], out_hbm.at[pl.ds(base + ci * _C, _C)],
            ssem[slot])

    for ci in range(_NBUF - 1):
        gather(ci).start()
    for ci in range(_NCHUNK):
        gather(ci).wait()
        scatter(ci).start()
        nxt = ci + _NBUF - 1
        if nxt < _NCHUNK:
            if ci >= 1:
                scatter(ci - 1).wait()
            gather(nxt).start()
    for ci in range(_NCHUNK - _NBUF, _NCHUNK):
        scatter(ci).wait()


def _flat_index_vector(num_rows):
    perm = np.asarray(jax.random.permutation(jax.random.key(42), num_rows))
    idx = perm[None, :] + np.arange(_B)[:, None] * num_rows
    return idx.reshape(-1).astype(np.int32)


_IDX = _flat_index_vector(_R)


def kernel(x):
    idx = jnp.asarray(_IDX)
    flat = x.reshape(_ROWS, _D)
    out = _shuffle_sc(flat, idx)
    return out.reshape(_B, _R, _D)
